# baseline (speedup 1.0000x reference)
import functools

import jax
import jax.numpy as jnp
from jax import lax
from jax.experimental import pallas as pl
from jax.experimental.pallas import tpu as pltpu
from jax.experimental.pallas import tpu_sc as plsc

NUM_CORES = 2
NUM_SUBCORES = 16
NW = NUM_CORES * NUM_SUBCORES

HIST = 50
EMB = 32
CB = 128
CB_PER_W = 4
ROWS_CB = CB * HIST
ROWS_Q = ROWS_CB // 2
QC = 64


@functools.lru_cache(maxsize=None)
def _build_detile(vocab: int, emb: int):
    n_full = vocab // 128
    tail = vocab - n_full * 128
    n_blk = vocab * emb // 1024
    mesh = plsc.VectorSubcoreMesh(
        core_axis_name="c", subcore_axis_name="s",
        num_cores=NUM_CORES, num_subcores=NUM_SUBCORES,
    )
    NBUF = 4
    iters = -(-n_full // NW)
    n_j = -(-iters // NBUF)

    @functools.partial(
        pl.kernel,
        out_type=jax.ShapeDtypeStruct((n_blk, 8, 128), jnp.float32),
        mesh=mesh,
        compiler_params=pltpu.CompilerParams(
            use_tc_tiling_on_sc=True, needs_layout_passes=False,
        ),
        scratch_types=(
            [pltpu.VMEM((emb, 129), jnp.float32) for _ in range(NBUF)]
            + [pltpu.VMEM((4, 8, 128), jnp.float32) for _ in range(NBUF)]
            + [pltpu.SemaphoreType.DMA] * (2 * NBUF)
        ),
    )
    def detile_kernel(tt_hbm, tail_hbm, out_hbm, *bufs):
        srcs = bufs[:NBUF]
        stgs = bufs[NBUF:2 * NBUF]
        ssems = bufs[2 * NBUF:3 * NBUF]
        wsems = bufs[3 * NBUF:4 * NBUF]
        wid = lax.axis_index("s") * NUM_CORES + lax.axis_index("c")

        def src_dma(t, tc):
            return pltpu.make_async_copy(
                tt_hbm.at[:, pl.ds(tc * 128, 128)],
                srcs[t].at[:, pl.ds(0, 128)],
                ssems[t],
            )

        def out_dma(t, tc, nb):
            return pltpu.make_async_copy(
                stgs[t].at[pl.ds(0, nb)],
                out_hbm.at[pl.ds(tc * 4, nb)],
                wsems[t],
            )

        def transpose(src, stg, nv):
            for v in range(nv):
                for half in range(emb // 16):
                    e_vec = lax.iota(jnp.int32, 16) + 16 * half
                    v_vec = jnp.broadcast_to(jnp.int32(v), (16,))
                    vals = plsc.load_gather(src, [e_vec, v_vec])
                    stg[v >> 5, (v >> 2) & 7,
                        pl.ds((v & 3) * 32 + half * 16, 16)] = vals

        for t in range(NBUF):
            tc = wid + t * NW

            @pl.when(tc < n_full)
            def _pre():
                src_dma(t, tc).start()

        def j_body(j, carry):
            for t in range(NBUF):
                tc = wid + (j * NBUF + t) * NW

                @pl.when(tc < n_full)
                def _do():
                    src_dma(t, tc).wait()

                    @pl.when(j >= 1)
                    def _wait_w():
                        out_dma(t, tc - NBUF * NW, 4).wait()

                    transpose(srcs[t], stgs[t], 128)
                    out_dma(t, tc, 4).start()
                    nxt = tc + NBUF * NW

                    @pl.when(nxt < n_full)
                    def _next():
                        src_dma(t, nxt).start()
            return carry

        lax.fori_loop(0, n_j, j_body, 0)
        for t in range(NBUF):
            last = wid + ((n_j - 1) * NBUF + t) * NW

            @pl.when(last < n_full)
            def _drain():
                out_dma(t, last, 4).wait()

        if tail:
            nb_tail = tail * emb // 1024

            @pl.when(wid == 0)
            def _tail():
                pltpu.sync_copy(tail_hbm, stgs[0].at[pl.ds(0, nb_tail)])
                pltpu.sync_copy(
                    stgs[0].at[pl.ds(0, nb_tail)],
                    out_hbm.at[pl.ds(n_full * 4, nb_tail)],
                )

    return detile_kernel


@functools.lru_cache(maxsize=None)
def _build(n_b: int):
    n_cb = n_b // CB
    assert n_cb == NW * CB_PER_W
    mesh = plsc.VectorSubcoreMesh(
        core_axis_name="c", subcore_axis_name="s",
        num_cores=NUM_CORES, num_subcores=NUM_SUBCORES,
    )

    @functools.partial(
        pl.kernel,
        out_type=jax.ShapeDtypeStruct((HIST, EMB // 8, n_cb, 8, CB), jnp.float32),
        mesh=mesh,
        compiler_params=pltpu.CompilerParams(
            use_tc_tiling_on_sc=False, needs_layout_passes=False,
        ),
        scratch_types=[
            pltpu.VMEM((ROWS_CB,), jnp.int32),
            pltpu.VMEM((ROWS_Q, EMB), jnp.float32),
            pltpu.VMEM((EMB // 8, 8, QC + 1), jnp.float32),
            pltpu.VMEM((EMB // 8, 8, QC + 1), jnp.float32),
            pltpu.SemaphoreType.DMA,
            pltpu.SemaphoreType.DMA,
            pltpu.SemaphoreType.DMA,
        ],
    )
    def emb_kernel(idx_hbm, table_hbm, out_hbm, idx_v, gbuf, stg0, stg1,
                   sem_g, sw0, sw1):
        wid = lax.axis_index("s") * NUM_CORES + lax.axis_index("c")
        stgs = (stg0, stg1)
        sems = (sw0, sw1)
        iota = lax.iota(jnp.int32, 16)
        tr0 = lax.shift_right_logical(iota, 3)
        tr1 = tr0 + 2
        rr = lax.bitwise_and(iota, 7)

        def cb_body(i, carry):
            cb = wid * CB_PER_W + i
            pltpu.sync_copy(idx_hbm.at[pl.ds(cb * ROWS_CB, ROWS_CB)], idx_v)
            for q in range(2):
                pltpu.async_copy(
                    table_hbm.at[idx_v.at[pl.ds(q * ROWS_Q, ROWS_Q)]],
                    gbuf, sem_g,
                ).wait()

                def h_body(hi, carry2):
                    for t in range(2):
                        hh = hi * 2 + t
                        stg = stgs[t]
                        sem = sems[t]

                        @pl.when(hi >= 1)
                        def _wait_prev():
                            pltpu.make_async_copy(
                                stg.at[:, :, pl.ds(0, QC)],
                                out_hbm.at[hh - 2, :, cb, :, pl.ds(q * QC, QC)],
                                sem,
                            ).wait()

                        for c in range(QC):
                            row = c * HIST + hh
                            cvec = jnp.broadcast_to(jnp.int32(c), (16,))
                            v0 = gbuf[row, pl.ds(0, 16)]
                            v1 = gbuf[row, pl.ds(16, 16)]
                            plsc.store_scatter(stg, [tr0, rr, cvec], v0)
                            plsc.store_scatter(stg, [tr1, rr, cvec], v1)
                        pltpu.async_copy(
                            stg.at[:, :, pl.ds(0, QC)],
                            out_hbm.at[hh, :, cb, :, pl.ds(q * QC, QC)],
                            sem,
                        )
                    return carry2

                lax.fori_loop(0, HIST // 2, h_body, 0)
                for t in range(2):
                    pltpu.make_async_copy(
                        stgs[t].at[:, :, pl.ds(0, QC)],
                        out_hbm.at[HIST - 2 + t, :, cb, :, pl.ds(q * QC, QC)],
                        sems[t],
                    ).wait()
            return carry

        lax.fori_loop(0, CB_PER_W, cb_body, 0)

    return emb_kernel


def kernel(x, table):
    b, h = x.shape
    vocab, emb = table.shape
    idx_flat = x.reshape(b * h).astype(jnp.int32)
    n_full = vocab // 128
    tail5 = table[n_full * 128:, :].reshape(-1, 8, 128)
    table_lin = _build_detile(vocab, emb)(table.T, tail5).reshape(vocab, emb)
    out5 = _build(b)(idx_flat, table_lin)
    return out5.transpose(2, 4, 0, 1, 3).reshape(b, h, emb)


# speedup vs baseline: 1.6146x; 1.6146x over previous
import functools

import jax
import jax.numpy as jnp
from jax import lax
from jax.experimental import pallas as pl
from jax.experimental.pallas import tpu as pltpu
from jax.experimental.pallas import tpu_sc as plsc

NUM_CORES = 2
NUM_SUBCORES = 16
NW = NUM_CORES * NUM_SUBCORES

HIST = 50
EMB = 32
CB = 128
CB_PER_W = 4
ROWS_CB = CB * HIST
ROWS_Q = ROWS_CB // 2
QC = 64


@functools.lru_cache(maxsize=None)
def _build_detile(vocab: int, emb: int):
    n_full = vocab // 128
    tail = vocab - n_full * 128
    n_blk = vocab * emb // 1024
    mesh = plsc.VectorSubcoreMesh(
        core_axis_name="c", subcore_axis_name="s",
        num_cores=NUM_CORES, num_subcores=NUM_SUBCORES,
    )
    NBUF = 4
    iters = -(-n_full // NW)
    n_j = -(-iters // NBUF)

    @functools.partial(
        pl.kernel,
        out_type=jax.ShapeDtypeStruct((n_blk, 8, 128), jnp.float32),
        mesh=mesh,
        compiler_params=pltpu.CompilerParams(
            use_tc_tiling_on_sc=True, needs_layout_passes=False,
        ),
        scratch_types=(
            [pltpu.VMEM((emb, 129), jnp.float32) for _ in range(NBUF)]
            + [pltpu.VMEM((4, 8, 128), jnp.float32) for _ in range(NBUF)]
            + [pltpu.SemaphoreType.DMA] * (2 * NBUF)
        ),
    )
    def detile_kernel(tt_hbm, tail_hbm, out_hbm, *bufs):
        srcs = bufs[:NBUF]
        stgs = bufs[NBUF:2 * NBUF]
        ssems = bufs[2 * NBUF:3 * NBUF]
        wsems = bufs[3 * NBUF:4 * NBUF]
        wid = lax.axis_index("s") * NUM_CORES + lax.axis_index("c")

        def src_dma(t, tc):
            return pltpu.make_async_copy(
                tt_hbm.at[:, pl.ds(tc * 128, 128)],
                srcs[t].at[:, pl.ds(0, 128)],
                ssems[t],
            )

        def out_dma(t, tc, nb):
            return pltpu.make_async_copy(
                stgs[t].at[pl.ds(0, nb)],
                out_hbm.at[pl.ds(tc * 4, nb)],
                wsems[t],
            )

        def transpose(src, stg, nv):
            for v0 in range(0, nv, 8):
                vals = []
                for v in range(v0, v0 + 8):
                    for half in range(emb // 16):
                        e_vec = lax.iota(jnp.int32, 16) + 16 * half
                        v_vec = jnp.broadcast_to(jnp.int32(v), (16,))
                        vals.append(plsc.load_gather(src, [e_vec, v_vec]))
                i = 0
                for v in range(v0, v0 + 8):
                    for half in range(emb // 16):
                        stg[v >> 5, (v >> 2) & 7,
                            pl.ds((v & 3) * 32 + half * 16, 16)] = vals[i]
                        i += 1

        for t in range(NBUF):
            tc = wid + t * NW

            @pl.when(tc < n_full)
            def _pre():
                src_dma(t, tc).start()

        def j_body(j, carry):
            for t in range(NBUF):
                tc = wid + (j * NBUF + t) * NW

                @pl.when(tc < n_full)
                def _do():
                    src_dma(t, tc).wait()

                    @pl.when(j >= 1)
                    def _wait_w():
                        out_dma(t, tc - NBUF * NW, 4).wait()

                    transpose(srcs[t], stgs[t], 128)
                    out_dma(t, tc, 4).start()
                    nxt = tc + NBUF * NW

                    @pl.when(nxt < n_full)
                    def _next():
                        src_dma(t, nxt).start()
            return carry

        lax.fori_loop(0, n_j, j_body, 0)
        for t in range(NBUF):
            last = wid + ((n_j - 1) * NBUF + t) * NW

            @pl.when(last < n_full)
            def _drain():
                out_dma(t, last, 4).wait()

        if tail:
            nb_tail = tail * emb // 1024

            @pl.when(wid == 0)
            def _tail():
                pltpu.sync_copy(tail_hbm, stgs[0].at[pl.ds(0, nb_tail)])
                pltpu.sync_copy(
                    stgs[0].at[pl.ds(0, nb_tail)],
                    out_hbm.at[pl.ds(n_full * 4, nb_tail)],
                )

    return detile_kernel


@functools.lru_cache(maxsize=None)
def _build(n_b: int):
    n_cb = n_b // CB
    assert n_cb == NW * CB_PER_W
    mesh = plsc.VectorSubcoreMesh(
        core_axis_name="c", subcore_axis_name="s",
        num_cores=NUM_CORES, num_subcores=NUM_SUBCORES,
    )

    @functools.partial(
        pl.kernel,
        out_type=jax.ShapeDtypeStruct((HIST, EMB // 8, n_cb, 8, CB), jnp.float32),
        mesh=mesh,
        compiler_params=pltpu.CompilerParams(
            use_tc_tiling_on_sc=False, needs_layout_passes=False,
        ),
        scratch_types=[
            pltpu.VMEM((ROWS_CB,), jnp.int32),
            pltpu.VMEM((ROWS_Q, EMB), jnp.float32),
            pltpu.VMEM((EMB // 8, 8, QC + 1), jnp.float32),
            pltpu.VMEM((EMB // 8, 8, QC + 1), jnp.float32),
            pltpu.SemaphoreType.DMA,
            pltpu.SemaphoreType.DMA,
            pltpu.SemaphoreType.DMA,
        ],
    )
    def emb_kernel(idx_hbm, table_hbm, out_hbm, idx_v, gbuf, stg0, stg1,
                   sem_g, sw0, sw1):
        wid = lax.axis_index("s") * NUM_CORES + lax.axis_index("c")
        stgs = (stg0, stg1)
        sems = (sw0, sw1)
        iota = lax.iota(jnp.int32, 16)
        tr0 = lax.shift_right_logical(iota, 3)
        tr1 = tr0 + 2
        rr = lax.bitwise_and(iota, 7)

        def cb_body(i, carry):
            cb = wid * CB_PER_W + i
            pltpu.sync_copy(idx_hbm.at[pl.ds(cb * ROWS_CB, ROWS_CB)], idx_v)
            for q in range(2):
                pltpu.async_copy(
                    table_hbm.at[idx_v.at[pl.ds(q * ROWS_Q, ROWS_Q)]],
                    gbuf, sem_g,
                ).wait()

                def h_body(hi, carry2):
                    for t in range(2):
                        hh = hi * 2 + t
                        stg = stgs[t]
                        sem = sems[t]

                        @pl.when(hi >= 1)
                        def _wait_prev():
                            pltpu.make_async_copy(
                                stg.at[:, :, pl.ds(0, QC)],
                                out_hbm.at[hh - 2, :, cb, :, pl.ds(q * QC, QC)],
                                sem,
                            ).wait()

                        for c0 in range(0, QC, 8):
                            vals = []
                            for c in range(c0, c0 + 8):
                                row = c * HIST + hh
                                vals.append(gbuf[row, pl.ds(0, 16)])
                                vals.append(gbuf[row, pl.ds(16, 16)])
                            for i, c in enumerate(range(c0, c0 + 8)):
                                cvec = jnp.broadcast_to(jnp.int32(c), (16,))
                                plsc.store_scatter(stg, [tr0, rr, cvec],
                                                   vals[2 * i])
                                plsc.store_scatter(stg, [tr1, rr, cvec],
                                                   vals[2 * i + 1])
                        pltpu.async_copy(
                            stg.at[:, :, pl.ds(0, QC)],
                            out_hbm.at[hh, :, cb, :, pl.ds(q * QC, QC)],
                            sem,
                        )
                    return carry2

                lax.fori_loop(0, HIST // 2, h_body, 0)
                for t in range(2):
                    pltpu.make_async_copy(
                        stgs[t].at[:, :, pl.ds(0, QC)],
                        out_hbm.at[HIST - 2 + t, :, cb, :, pl.ds(q * QC, QC)],
                        sems[t],
                    ).wait()
            return carry

        lax.fori_loop(0, CB_PER_W, cb_body, 0)

    return emb_kernel


def kernel(x, table):
    b, h = x.shape
    vocab, emb = table.shape
    idx_flat = x.reshape(b * h).astype(jnp.int32)
    n_full = vocab // 128
    tail5 = table[n_full * 128:, :].reshape(-1, 8, 128)
    table_lin = _build_detile(vocab, emb)(table.T, tail5).reshape(vocab, emb)
    out5 = _build(b)(idx_flat, table_lin)
    return out5.transpose(2, 4, 0, 1, 3).reshape(b, h, emb)


# speedup vs baseline: 1.6598x; 1.0280x over previous
import functools

import jax
import jax.numpy as jnp
from jax import lax
from jax.experimental import pallas as pl
from jax.experimental.pallas import tpu as pltpu
from jax.experimental.pallas import tpu_sc as plsc

NUM_CORES = 2
NUM_SUBCORES = 16
NW = NUM_CORES * NUM_SUBCORES

HIST = 50
EMB = 32
CB = 128
CB_PER_W = 4
ROWS_CB = CB * HIST
ROWS_Q = ROWS_CB // 2
QC = 64


@functools.lru_cache(maxsize=None)
def _build_detile(vocab: int, emb: int):
    n_full = vocab // 128
    tail = vocab - n_full * 128
    n_blk = vocab * emb // 1024
    BC = 4
    PW = BC * 128
    n_batch = n_full // BC
    mesh = plsc.VectorSubcoreMesh(
        core_axis_name="c", subcore_axis_name="s",
        num_cores=NUM_CORES, num_subcores=NUM_SUBCORES,
    )
    NBUF = 2
    n_j = -(-(-(-n_batch // NW)) // NBUF)

    @functools.partial(
        pl.kernel,
        out_type=jax.ShapeDtypeStruct((n_blk, 8, 128), jnp.float32),
        mesh=mesh,
        compiler_params=pltpu.CompilerParams(
            use_tc_tiling_on_sc=True, needs_layout_passes=False,
        ),
        scratch_types=(
            [pltpu.VMEM((emb, PW + 1), jnp.float32) for _ in range(NBUF)]
            + [pltpu.VMEM((BC * 4, 8, 128), jnp.float32) for _ in range(NBUF)]
            + [pltpu.SemaphoreType.DMA] * (2 * NBUF)
        ),
    )
    def detile_kernel(tt_hbm, tail_hbm, out_hbm, *bufs):
        srcs = bufs[:NBUF]
        stgs = bufs[NBUF:2 * NBUF]
        ssems = bufs[2 * NBUF:3 * NBUF]
        wsems = bufs[3 * NBUF:4 * NBUF]
        wid = lax.axis_index("s") * NUM_CORES + lax.axis_index("c")

        def src_dma(t, b):
            return pltpu.make_async_copy(
                tt_hbm.at[:, pl.ds(b * PW, PW)],
                srcs[t].at[:, pl.ds(0, PW)],
                ssems[t],
            )

        def out_dma(t, b):
            return pltpu.make_async_copy(
                stgs[t], out_hbm.at[pl.ds(b * BC * 4, BC * 4)], wsems[t],
            )

        e_half = [lax.iota(jnp.int32, 16) + 16 * h for h in range(emb // 16)]

        def transpose(src, stg, np_):
            def p_body(pi, carry):
                p0 = pi * 8
                vals = []
                for k in range(8):
                    p_vec = jnp.broadcast_to(p0 + k, (16,))
                    for h in range(emb // 16):
                        vals.append(plsc.load_gather(src, [e_half[h], p_vec]))
                i = 0
                for k in range(8):
                    p = p0 + k
                    for h in range(emb // 16):
                        stg[p >> 5, (p >> 2) & 7,
                            pl.ds((p & 3) * 32 + h * 16, 16)] = vals[i]
                        i += 1
                return carry

            lax.fori_loop(0, np_ // 8, p_body, 0)

        for t in range(NBUF):
            b = wid + t * NW

            @pl.when(b < n_batch)
            def _pre():
                src_dma(t, b).start()

        def j_body(j, carry):
            for t in range(NBUF):
                b = wid + (j * NBUF + t) * NW
                bp = b - NBUF * NW

                @pl.when(jnp.logical_and(j >= 1, bp < n_batch))
                def _wait_prev():
                    out_dma(t, bp).wait()

                @pl.when(b < n_batch)
                def _do():
                    src_dma(t, b).wait()
                    transpose(srcs[t], stgs[t], PW)
                    out_dma(t, b).start()
                    nxt = b + NBUF * NW

                    @pl.when(nxt < n_batch)
                    def _next():
                        src_dma(t, nxt).start()
            return carry

        lax.fori_loop(0, n_j, j_body, 0)
        for t in range(NBUF):
            b_last = wid + ((n_j - 1) * NBUF + t) * NW

            @pl.when(b_last < n_batch)
            def _drain():
                out_dma(t, b_last).wait()

        if tail:
            nb_tail = tail * emb // 1024

            @pl.when(wid == 0)
            def _tail():
                pltpu.sync_copy(tail_hbm, stgs[0].at[pl.ds(0, nb_tail)])
                pltpu.sync_copy(
                    stgs[0].at[pl.ds(0, nb_tail)],
                    out_hbm.at[pl.ds(n_full * 4, nb_tail)],
                )

    return detile_kernel


@functools.lru_cache(maxsize=None)
def _build(n_b: int):
    n_cb = n_b // CB
    assert n_cb == NW * CB_PER_W
    mesh = plsc.VectorSubcoreMesh(
        core_axis_name="c", subcore_axis_name="s",
        num_cores=NUM_CORES, num_subcores=NUM_SUBCORES,
    )

    @functools.partial(
        pl.kernel,
        out_type=jax.ShapeDtypeStruct((HIST, EMB // 8, n_cb, 8, CB), jnp.float32),
        mesh=mesh,
        compiler_params=pltpu.CompilerParams(
            use_tc_tiling_on_sc=False, needs_layout_passes=False,
        ),
        scratch_types=[
            pltpu.VMEM((ROWS_CB,), jnp.int32),
            pltpu.VMEM((ROWS_Q, EMB), jnp.float32),
            pltpu.VMEM((EMB // 8, 8, QC + 1), jnp.float32),
            pltpu.VMEM((EMB // 8, 8, QC + 1), jnp.float32),
            pltpu.SemaphoreType.DMA,
            pltpu.SemaphoreType.DMA,
            pltpu.SemaphoreType.DMA,
        ],
    )
    def emb_kernel(idx_hbm, table_hbm, out_hbm, idx_v, gbuf, stg0, stg1,
                   sem_g, sw0, sw1):
        wid = lax.axis_index("s") * NUM_CORES + lax.axis_index("c")
        stgs = (stg0, stg1)
        sems = (sw0, sw1)
        iota = lax.iota(jnp.int32, 16)
        tr0 = lax.shift_right_logical(iota, 3)
        tr1 = tr0 + 2
        rr = lax.bitwise_and(iota, 7)

        def cb_body(i, carry):
            cb = wid * CB_PER_W + i
            pltpu.sync_copy(idx_hbm.at[pl.ds(cb * ROWS_CB, ROWS_CB)], idx_v)
            for q in range(2):
                pltpu.async_copy(
                    table_hbm.at[idx_v.at[pl.ds(q * ROWS_Q, ROWS_Q)]],
                    gbuf, sem_g,
                ).wait()

                def h_body(hi, carry2):
                    for t in range(2):
                        hh = hi * 2 + t
                        stg = stgs[t]
                        sem = sems[t]

                        @pl.when(hi >= 1)
                        def _wait_prev():
                            pltpu.make_async_copy(
                                stg.at[:, :, pl.ds(0, QC)],
                                out_hbm.at[hh - 2, :, cb, :, pl.ds(q * QC, QC)],
                                sem,
                            ).wait()

                        for c0 in range(0, QC, 8):
                            vals = []
                            for c in range(c0, c0 + 8):
                                row = c * HIST + hh
                                vals.append(gbuf[row, pl.ds(0, 16)])
                                vals.append(gbuf[row, pl.ds(16, 16)])
                            for i, c in enumerate(range(c0, c0 + 8)):
                                cvec = jnp.broadcast_to(jnp.int32(c), (16,))
                                plsc.store_scatter(stg, [tr0, rr, cvec],
                                                   vals[2 * i])
                                plsc.store_scatter(stg, [tr1, rr, cvec],
                                                   vals[2 * i + 1])
                        pltpu.async_copy(
                            stg.at[:, :, pl.ds(0, QC)],
                            out_hbm.at[hh, :, cb, :, pl.ds(q * QC, QC)],
                            sem,
                        )
                    return carry2

                lax.fori_loop(0, HIST // 2, h_body, 0)
                for t in range(2):
                    pltpu.make_async_copy(
                        stgs[t].at[:, :, pl.ds(0, QC)],
                        out_hbm.at[HIST - 2 + t, :, cb, :, pl.ds(q * QC, QC)],
                        sems[t],
                    ).wait()
            return carry

        lax.fori_loop(0, CB_PER_W, cb_body, 0)

    return emb_kernel


def kernel(x, table):
    b, h = x.shape
    vocab, emb = table.shape
    idx_flat = x.reshape(b * h).astype(jnp.int32)
    n_full = vocab // 128
    tail5 = table[n_full * 128:, :].reshape(-1, 8, 128)
    table_lin = _build_detile(vocab, emb)(table.T, tail5).reshape(vocab, emb)
    out5 = _build(b)(idx_flat, table_lin)
    return out5.transpose(2, 4, 0, 1, 3).reshape(b, h, emb)
